# baseline (speedup 1.0000x reference)
import jax
import jax.numpy as jnp
from jax import lax
from jax.experimental import pallas as pl
from jax.experimental.pallas import tpu as pltpu
from jax.experimental.pallas import tpu_sc as plsc

_BATCH = 16384
_D = 32
_NW = 32
_BPW = _BATCH // _NW
_NBLK = _BPW // 16


def _sc_body(users_hbm, product_hbm, utab_hbm, ptab_hbm, wb_hbm, bias_hbm,
             out_hbm, uidx_v, pidx_v, urows_v, prows_v, wb_v, bias_v, out_v,
             sem_u, sem_p):
    c = lax.axis_index("c")
    s = lax.axis_index("s")
    wid = s * 2 + c
    base = wid * _BPW

    pltpu.sync_copy(users_hbm.at[pl.ds(base, _BPW)], uidx_v)
    pltpu.sync_copy(product_hbm.at[pl.ds(base, _BPW)], pidx_v)

    cp_u = pltpu.async_copy(utab_hbm.at[uidx_v], urows_v, sem_u)
    cp_p = pltpu.async_copy(ptab_hbm.at[pidx_v], prows_v, sem_p)

    pltpu.sync_copy(wb_hbm, wb_v)
    pltpu.sync_copy(bias_hbm, bias_v)
    bias = bias_v[...]

    cp_u.wait()
    cp_p.wait()

    def blk(j, carry):
        row_ids = j * 16 + lax.iota(jnp.int32, 16)
        acc = bias
        for d in range(_D):
            col = jnp.full((16,), d, jnp.int32)
            acc = acc + plsc.load_gather(urows_v, [row_ids, col]) * wb_v[d, :]
        for d in range(_D):
            col = jnp.full((16,), d, jnp.int32)
            acc = acc + plsc.load_gather(prows_v, [row_ids, col]) * wb_v[_D + d, :]
        out_v[pl.ds(j * 16, 16)] = acc
        return carry

    lax.fori_loop(0, _NBLK, blk, 0)
    pltpu.sync_copy(out_v, out_hbm.at[pl.ds(base, _BPW)])


@jax.jit
def _run(users, product, user_table, product_table, W, b):
    wb = jnp.broadcast_to(W.reshape(2 * _D, 1), (2 * _D, 16))
    bb = jnp.broadcast_to(b.reshape(1), (16,))
    mesh = plsc.VectorSubcoreMesh(core_axis_name="c", subcore_axis_name="s")
    out = pl.kernel(
        _sc_body,
        mesh=mesh,
        out_type=jax.ShapeDtypeStruct((_BATCH,), jnp.float32),
        scratch_types=[
            pltpu.VMEM((_BPW,), jnp.int32),
            pltpu.VMEM((_BPW,), jnp.int32),
            pltpu.VMEM((_BPW, _D), jnp.float32),
            pltpu.VMEM((_BPW, _D), jnp.float32),
            pltpu.VMEM((2 * _D, 16), jnp.float32),
            pltpu.VMEM((16,), jnp.float32),
            pltpu.VMEM((_BPW,), jnp.float32),
            pltpu.SemaphoreType.DMA,
            pltpu.SemaphoreType.DMA,
        ],
        compiler_params=pltpu.CompilerParams(
            needs_layout_passes=False, use_tc_tiling_on_sc=False),
    )(users, product, user_table, product_table, wb, bb)
    return out.reshape(_BATCH, 1)


def kernel(users, product, user_table, product_table, W, b):
    return _run(users, product, user_table, product_table, W, b)
